# baseline (speedup 1.0000x reference)
import functools

import jax
import jax.numpy as jnp
from jax import lax
from jax.experimental import pallas as pl
from jax.experimental.pallas import tpu as pltpu
from jax.experimental.pallas import tpu_sc as plsc

VOCAB = 100000
EMB = 64
MAX_NORM = 1.0
B = 1024
L = 20
N = B * L
CHUNK = 128
NCHUNKS = N // CHUNK

_info = plsc.get_sparse_core_info()
NC, NS = _info.num_cores, _info.num_subcores
NW = NC * NS
CPW = NCHUNKS // NW

TN = 2048
GRID = (VOCAB + TN - 1) // TN


def _sc_gather_body(table_hbm, idx_hbm, out_hbm, idx_v, rows_v, sem):
    wid = lax.axis_index("s") * NC + lax.axis_index("c")
    base = wid * CPW
    for j in range(CPW):
        pltpu.sync_copy(
            idx_hbm.at[pl.ds((base + j) * CHUNK, CHUNK)], idx_v.at[j])
    copies = [
        pltpu.async_copy(table_hbm.at[idx_v.at[j]], rows_v.at[j], sem)
        for j in range(CPW)
    ]
    for c in copies:
        c.wait()
    pltpu.sync_copy(rows_v, out_hbm.at[pl.ds(base, CPW)])


_sc_gather = functools.partial(
    pl.kernel,
    mesh=plsc.VectorSubcoreMesh(core_axis_name="c", subcore_axis_name="s"),
    out_type=jax.ShapeDtypeStruct((NCHUNKS, CHUNK, EMB), jnp.float32),
    scratch_types=[
        pltpu.VMEM((CPW, CHUNK), jnp.int32),
        pltpu.VMEM((CPW, CHUNK, EMB), jnp.float32),
        pltpu.SemaphoreType.DMA,
    ],
    compiler_params=pltpu.CompilerParams(use_tc_tiling_on_sc=False),
)(_sc_gather_body)


def _mm_body(vecs_ref, w_ref, b_ref, out_ref, h_ref):
    @pl.when(pl.program_id(0) == 0)
    def _():
        v = vecs_ref[...]
        ss = jnp.sum(v * v, axis=-1, keepdims=True)
        norm = jnp.sqrt(ss)
        scale = jnp.where(norm > MAX_NORM, MAX_NORM / (norm + 1e-7), 1.0)
        h_ref[...] = jnp.sum(v * scale, axis=0) * (1.0 / L)

    w = w_ref[...]
    out_ref[...] = lax.dot_general(
        h_ref[...], w,
        dimension_numbers=(((1,), (1,)), ((), ())),
        preferred_element_type=jnp.float32,
    ) + b_ref[...]


_mm = pl.pallas_call(
    _mm_body,
    grid=(GRID,),
    in_specs=[
        pl.BlockSpec((L, B, EMB), lambda i: (0, 0, 0)),
        pl.BlockSpec((TN, EMB), lambda i: (i, 0)),
        pl.BlockSpec((1, TN), lambda i: (0, i)),
    ],
    out_specs=pl.BlockSpec((B, TN), lambda i: (0, i)),
    out_shape=jax.ShapeDtypeStruct((B, VOCAB), jnp.float32),
    scratch_shapes=[pltpu.VMEM((B, EMB), jnp.float32)],
    compiler_params=pltpu.CompilerParams(
        dimension_semantics=("arbitrary",),
    ),
)


def kernel(inputs, emb_table, W, b):
    idx = inputs.T.reshape(N)
    vecs = _sc_gather(emb_table, idx)
    vecs3 = vecs.reshape(L, B, EMB)
    return _mm(vecs3, W, b.reshape(1, VOCAB))


# speedup vs baseline: 2.6573x; 2.6573x over previous
import functools

import jax
import jax.numpy as jnp
from jax import lax
from jax.experimental import pallas as pl
from jax.experimental.pallas import tpu as pltpu
from jax.experimental.pallas import tpu_sc as plsc

VOCAB = 100000
EMB = 64
MAX_NORM = 1.0
B = 1024
L = 20
N = B * L
CHUNK = 128
NCHUNKS = N // CHUNK

_info = plsc.get_sparse_core_info()
NC, NS = _info.num_cores, _info.num_subcores
NW = NC * NS
CPW = NCHUNKS // NW

TN = 2048
GRID = (VOCAB + TN - 1) // TN


def _sc_gather_body(table_hbm, idx_hbm, out_hbm, idx_v, rows_v, sem):
    wid = lax.axis_index("s") * NC + lax.axis_index("c")
    base = wid * CPW
    for j in range(CPW):
        pltpu.sync_copy(
            idx_hbm.at[pl.ds((base + j) * CHUNK, CHUNK)], idx_v.at[j])
    copies = [
        pltpu.async_copy(table_hbm.at[idx_v.at[j]], rows_v.at[j], sem)
        for j in range(CPW)
    ]
    for c in copies:
        c.wait()
    pltpu.sync_copy(rows_v, out_hbm.at[pl.ds(base, CPW)])


_sc_gather = functools.partial(
    pl.kernel,
    mesh=plsc.VectorSubcoreMesh(core_axis_name="c", subcore_axis_name="s"),
    out_type=jax.ShapeDtypeStruct((NCHUNKS, CHUNK, EMB), jnp.float32),
    scratch_types=[
        pltpu.VMEM((CPW, CHUNK), jnp.int32),
        pltpu.VMEM((CPW, CHUNK, EMB), jnp.float32),
        pltpu.SemaphoreType.DMA,
    ],
    compiler_params=pltpu.CompilerParams(use_tc_tiling_on_sc=False),
)(_sc_gather_body)


def _mm_body(vecs_ref, wt_ref, b_ref, out_ref, ht_ref):
    @pl.when(pl.program_id(0) == 0)
    def _():
        v = vecs_ref[...]
        ss = jnp.sum(v * v, axis=-1, keepdims=True)
        norm = jnp.sqrt(ss)
        scale = jnp.where(norm > MAX_NORM, MAX_NORM / (norm + 1e-7), 1.0)
        h = jnp.sum(v * scale, axis=0) * (1.0 / L)
        ht_ref[...] = h.T

    wt = wt_ref[...]
    res = lax.dot_general(
        wt, ht_ref[...],
        dimension_numbers=(((0,), (0,)), ((), ())),
        preferred_element_type=jnp.float32,
    )
    out_ref[...] = res + b_ref[...].T


_mm = pl.pallas_call(
    _mm_body,
    grid=(GRID,),
    in_specs=[
        pl.BlockSpec((L, B, EMB), lambda i: (0, 0, 0)),
        pl.BlockSpec((EMB, TN), lambda i: (0, i)),
        pl.BlockSpec((1, TN), lambda i: (0, i)),
    ],
    out_specs=pl.BlockSpec((TN, B), lambda i: (i, 0)),
    out_shape=jax.ShapeDtypeStruct((VOCAB, B), jnp.float32),
    scratch_shapes=[pltpu.VMEM((EMB, B), jnp.float32)],
    compiler_params=pltpu.CompilerParams(
        dimension_semantics=("arbitrary",),
    ),
)


def kernel(inputs, emb_table, W, b):
    idx = inputs.T.reshape(N)
    vecs = _sc_gather(emb_table, idx)
    vecs3 = vecs.reshape(L, B, EMB)
    logits_t = _mm(vecs3, W.T, b.reshape(1, VOCAB))
    return logits_t.T


# speedup vs baseline: 2.8979x; 1.0905x over previous
import functools

import jax
import jax.numpy as jnp
from jax import lax
from jax.experimental import pallas as pl
from jax.experimental.pallas import tpu as pltpu
from jax.experimental.pallas import tpu_sc as plsc

VOCAB = 100000
EMB = 64
PAD = 128
MAX_NORM = 1.0
B = 1024
L = 20
N = B * L
CHUNK = 128
NCHUNKS = N // CHUNK

_info = plsc.get_sparse_core_info()
NC, NS = _info.num_cores, _info.num_subcores
NW = NC * NS
CPW = NCHUNKS // NW

TN = 2048
GRID = (VOCAB + TN - 1) // TN


def _sc_gather_body(table_hbm, idx_hbm, out_hbm, idx_v, rows_v, sem):
    wid = lax.axis_index("s") * NC + lax.axis_index("c")
    base = wid * CPW
    for j in range(CPW):
        pltpu.sync_copy(
            idx_hbm.at[pl.ds((base + j) * CHUNK, CHUNK)], idx_v.at[j])
    copies = [
        pltpu.async_copy(table_hbm.at[idx_v.at[j]], rows_v.at[j], sem)
        for j in range(CPW)
    ]
    for c in copies:
        c.wait()
    pltpu.sync_copy(rows_v, out_hbm.at[pl.ds(base, CPW)])


_sc_gather = functools.partial(
    pl.kernel,
    mesh=plsc.VectorSubcoreMesh(core_axis_name="c", subcore_axis_name="s"),
    out_type=jax.ShapeDtypeStruct((NCHUNKS, CHUNK, PAD), jnp.float32),
    scratch_types=[
        pltpu.VMEM((CPW, CHUNK), jnp.int32),
        pltpu.VMEM((CPW, CHUNK, PAD), jnp.float32),
        pltpu.SemaphoreType.DMA,
    ],
    compiler_params=pltpu.CompilerParams(use_tc_tiling_on_sc=False),
)(_sc_gather_body)


def _tp_body(embt_ref, out_ref):
    vt = embt_ref[...].T
    z = jnp.zeros((TV, PAD - EMB), jnp.float32)
    out_ref[...] = jnp.concatenate([vt, z], axis=1)


TV = 2048
_tp = pl.pallas_call(
    _tp_body,
    grid=((VOCAB + TV - 1) // TV,),
    in_specs=[pl.BlockSpec((EMB, TV), lambda i: (0, i))],
    out_specs=pl.BlockSpec((TV, PAD), lambda i: (i, 0)),
    out_shape=jax.ShapeDtypeStruct((VOCAB, PAD), jnp.float32),
)


def _mm_body(vecs_ref, wt_ref, b_ref, out_ref, ht_ref):
    @pl.when(pl.program_id(0) == 0)
    def _():
        v = vecs_ref[...]
        ss = jnp.sum(v * v, axis=-1, keepdims=True)
        norm = jnp.sqrt(ss)
        scale = jnp.where(norm > MAX_NORM, MAX_NORM / (norm + 1e-7), 1.0)
        h = jnp.sum(v * scale, axis=0) * (1.0 / L)
        ht = h.T[:EMB, :]
        hhi = ht.astype(jnp.bfloat16)
        hlo = (ht - hhi.astype(jnp.float32)).astype(jnp.bfloat16)
        ht_ref[...] = jnp.concatenate([hhi, hhi, hlo], axis=0)

    wt = wt_ref[...]
    whi = wt.astype(jnp.bfloat16)
    wlo = (wt - whi.astype(jnp.float32)).astype(jnp.bfloat16)
    wcat = jnp.concatenate([whi, wlo, whi], axis=0)
    res = lax.dot_general(
        wcat, ht_ref[...],
        dimension_numbers=(((0,), (0,)), ((), ())),
        preferred_element_type=jnp.float32,
    )
    out_ref[...] = res + b_ref[...].T


_mm = pl.pallas_call(
    _mm_body,
    grid=(GRID,),
    in_specs=[
        pl.BlockSpec((L, B, PAD), lambda i: (0, 0, 0)),
        pl.BlockSpec((EMB, TN), lambda i: (0, i)),
        pl.BlockSpec((1, TN), lambda i: (0, i)),
    ],
    out_specs=pl.BlockSpec((TN, B), lambda i: (i, 0)),
    out_shape=jax.ShapeDtypeStruct((VOCAB, B), jnp.float32),
    scratch_shapes=[pltpu.VMEM((3 * EMB, B), jnp.bfloat16)],
    compiler_params=pltpu.CompilerParams(
        dimension_semantics=("arbitrary",),
    ),
)


def kernel(inputs, emb_table, W, b):
    table128 = _tp(emb_table.T)
    idx = inputs.T.reshape(N)
    vecs = _sc_gather(table128, idx)
    vecs3 = vecs.reshape(L, B, PAD)
    logits_t = _mm(vecs3, W.T, b.reshape(1, VOCAB))
    return logits_t.T


# speedup vs baseline: 2.9704x; 1.0250x over previous
import functools

import jax
import jax.numpy as jnp
from jax import lax
from jax.experimental import pallas as pl
from jax.experimental.pallas import tpu as pltpu
from jax.experimental.pallas import tpu_sc as plsc

VOCAB = 100000
EMB = 64
PAD = 128
MAX_NORM = 1.0
B = 1024
L = 20
N = B * L
CHUNK = 128
NCHUNKS = N // CHUNK

_info = plsc.get_sparse_core_info()
NC, NS = _info.num_cores, _info.num_subcores
NW = NC * NS
CPW = NCHUNKS // NW

TN = 2048
GRID = (VOCAB + TN - 1) // TN


def _sc_gather_body(table_hbm, idx_hbm, out_hbm, idx_v, rows_v, sem):
    wid = lax.axis_index("s") * NC + lax.axis_index("c")
    base = wid * CPW
    for j in range(CPW):
        pltpu.sync_copy(
            idx_hbm.at[pl.ds((base + j) * CHUNK, CHUNK)], idx_v.at[j])
    copies = [
        pltpu.async_copy(table_hbm.at[idx_v.at[j]], rows_v.at[j], sem)
        for j in range(CPW)
    ]
    for c in copies:
        c.wait()
    pltpu.sync_copy(rows_v, out_hbm.at[pl.ds(base, CPW)])


_sc_gather = functools.partial(
    pl.kernel,
    mesh=plsc.VectorSubcoreMesh(core_axis_name="c", subcore_axis_name="s"),
    out_type=jax.ShapeDtypeStruct((NCHUNKS, CHUNK, PAD), jnp.float32),
    scratch_types=[
        pltpu.VMEM((CPW, CHUNK), jnp.int32),
        pltpu.VMEM((CPW, CHUNK, PAD), jnp.float32),
        pltpu.SemaphoreType.DMA,
    ],
    compiler_params=pltpu.CompilerParams(use_tc_tiling_on_sc=False),
)(_sc_gather_body)


def _tp_body(embt_ref, out_ref):
    v = embt_ref[...]
    ss = jnp.sum(v * v, axis=0, keepdims=True)
    norm = jnp.sqrt(ss)
    scale = jnp.where(norm > MAX_NORM, MAX_NORM / (norm + 1e-7), 1.0)
    v = v * scale
    vhi = v.astype(jnp.bfloat16)
    vlo = (v - vhi.astype(jnp.float32)).astype(jnp.bfloat16)
    eye = (lax.broadcasted_iota(jnp.int32, (EMB, EMB), 0)
           == lax.broadcasted_iota(jnp.int32, (EMB, EMB), 1))
    ident = eye.astype(jnp.bfloat16)
    vt = lax.dot_general(
        jnp.concatenate([vhi, vlo], axis=0),
        jnp.concatenate([ident, ident], axis=0),
        dimension_numbers=(((0,), (0,)), ((), ())),
        preferred_element_type=jnp.float32,
    )
    z = jnp.zeros((TV, PAD - EMB), jnp.float32)
    out_ref[...] = jnp.concatenate([vt, z], axis=1)


TV = 2048
_tp = pl.pallas_call(
    _tp_body,
    grid=((VOCAB + TV - 1) // TV,),
    in_specs=[pl.BlockSpec((EMB, TV), lambda i: (0, i))],
    out_specs=pl.BlockSpec((TV, PAD), lambda i: (i, 0)),
    out_shape=jax.ShapeDtypeStruct((VOCAB, PAD), jnp.float32),
)


def _mm_body(vecs_ref, wt_ref, b_ref, out_ref, ht_ref):
    @pl.when(pl.program_id(0) == 0)
    def _():
        v = vecs_ref[...]
        h = jnp.sum(v, axis=0) * (1.0 / L)
        ht = h.T[:EMB, :]
        hhi = ht.astype(jnp.bfloat16)
        hlo = (ht - hhi.astype(jnp.float32)).astype(jnp.bfloat16)
        ht_ref[...] = jnp.concatenate([hhi, hhi, hlo], axis=0)

    wt = wt_ref[...]
    whi = wt.astype(jnp.bfloat16)
    wlo = (wt - whi.astype(jnp.float32)).astype(jnp.bfloat16)
    wcat = jnp.concatenate([whi, wlo, whi], axis=0)
    res = lax.dot_general(
        wcat, ht_ref[...],
        dimension_numbers=(((0,), (0,)), ((), ())),
        preferred_element_type=jnp.float32,
    )
    out_ref[...] = res + b_ref[...].T


_mm = pl.pallas_call(
    _mm_body,
    grid=(GRID,),
    in_specs=[
        pl.BlockSpec((L, B, PAD), lambda i: (0, 0, 0)),
        pl.BlockSpec((EMB, TN), lambda i: (0, i)),
        pl.BlockSpec((1, TN), lambda i: (0, i)),
    ],
    out_specs=pl.BlockSpec((TN, B), lambda i: (i, 0)),
    out_shape=jax.ShapeDtypeStruct((VOCAB, B), jnp.float32),
    scratch_shapes=[pltpu.VMEM((3 * EMB, B), jnp.bfloat16)],
    compiler_params=pltpu.CompilerParams(
        dimension_semantics=("arbitrary",),
    ),
)


def kernel(inputs, emb_table, W, b):
    table128 = _tp(emb_table.T)
    idx = inputs.T.reshape(N)
    vecs = _sc_gather(table128, idx)
    vecs3 = vecs.reshape(L, B, PAD)
    logits_t = _mm(vecs3, W.T, b.reshape(1, VOCAB))
    return logits_t.T


# speedup vs baseline: 3.2663x; 1.0996x over previous
import functools

import jax
import jax.numpy as jnp
from jax import lax
from jax.experimental import pallas as pl
from jax.experimental.pallas import tpu as pltpu
from jax.experimental.pallas import tpu_sc as plsc

VOCAB = 100000
EMB = 64
PAD = 128
MAX_NORM = 1.0
B = 1024
L = 20
N = B * L
CHUNK = 128
NCHUNKS = N // CHUNK

_info = plsc.get_sparse_core_info()
NC, NS = _info.num_cores, _info.num_subcores
NW = NC * NS
CPW = NCHUNKS // NW

TN = 2048
GRID = (VOCAB + TN - 1) // TN


def _sc_gather_body(table_hbm, idx_hbm, out_hbm, idx_v, rows_v, sem):
    wid = lax.axis_index("s") * NC + lax.axis_index("c")
    base = wid * CPW
    for j in range(CPW):
        pltpu.sync_copy(
            idx_hbm.at[pl.ds((base + j) * CHUNK, CHUNK)], idx_v.at[j])
    copies = [
        pltpu.async_copy(table_hbm.at[idx_v.at[j]], rows_v.at[j], sem)
        for j in range(CPW)
    ]
    for c in copies:
        c.wait()
    pltpu.sync_copy(rows_v, out_hbm.at[pl.ds(base, CPW)])


_sc_gather = functools.partial(
    pl.kernel,
    mesh=plsc.VectorSubcoreMesh(core_axis_name="c", subcore_axis_name="s"),
    out_type=jax.ShapeDtypeStruct((NCHUNKS, CHUNK, PAD), jnp.float32),
    scratch_types=[
        pltpu.VMEM((CPW, CHUNK), jnp.int32),
        pltpu.VMEM((CPW, CHUNK, PAD), jnp.float32),
        pltpu.SemaphoreType.DMA,
    ],
    compiler_params=pltpu.CompilerParams(use_tc_tiling_on_sc=False),
)(_sc_gather_body)


def _tp_body(embt_ref, out_ref):
    v = embt_ref[...]
    ss = jnp.sum(v * v, axis=0, keepdims=True)
    norm = jnp.sqrt(ss)
    scale = jnp.where(norm > MAX_NORM, MAX_NORM / (norm + 1e-7), 1.0)
    v = v * scale
    vhi = v.astype(jnp.bfloat16)
    vlo = (v - vhi.astype(jnp.float32)).astype(jnp.bfloat16)
    eye = (lax.broadcasted_iota(jnp.int32, (EMB, EMB), 0)
           == lax.broadcasted_iota(jnp.int32, (EMB, EMB), 1))
    ident = eye.astype(jnp.bfloat16)
    vt = lax.dot_general(
        jnp.concatenate([vhi, vlo], axis=0),
        jnp.concatenate([ident, ident], axis=0),
        dimension_numbers=(((0,), (0,)), ((), ())),
        preferred_element_type=jnp.float32,
    )
    z = jnp.zeros((TV, PAD - EMB), jnp.float32)
    out_ref[...] = jnp.concatenate([vt, z], axis=1)


TV = 8192
_tp = pl.pallas_call(
    _tp_body,
    grid=((VOCAB + TV - 1) // TV,),
    in_specs=[pl.BlockSpec((EMB, TV), lambda i: (0, i))],
    out_specs=pl.BlockSpec((TV, PAD), lambda i: (i, 0)),
    out_shape=jax.ShapeDtypeStruct((VOCAB, PAD), jnp.float32),
)


def _mm_body(vecs_ref, wt_ref, b_ref, out_ref, ht_ref):
    @pl.when(pl.program_id(0) == 0)
    def _():
        v = vecs_ref[...]
        h = jnp.sum(v, axis=0) * (1.0 / L)
        ht = h.T[:EMB, :]
        hhi = ht.astype(jnp.bfloat16)
        hlo = (ht - hhi.astype(jnp.float32)).astype(jnp.bfloat16)
        ht_ref[...] = jnp.concatenate([hhi, hhi, hlo], axis=0)

    wt = wt_ref[...]
    whi = wt.astype(jnp.bfloat16)
    wlo = (wt - whi.astype(jnp.float32)).astype(jnp.bfloat16)
    wcat = jnp.concatenate([whi, wlo, whi], axis=0)
    res = lax.dot_general(
        wcat, ht_ref[...],
        dimension_numbers=(((0,), (0,)), ((), ())),
        preferred_element_type=jnp.float32,
    )
    out_ref[...] = res + b_ref[...].T


_mm = pl.pallas_call(
    _mm_body,
    grid=(GRID,),
    in_specs=[
        pl.BlockSpec((L, B, PAD), lambda i: (0, 0, 0)),
        pl.BlockSpec((EMB, TN), lambda i: (0, i)),
        pl.BlockSpec((1, TN), lambda i: (0, i)),
    ],
    out_specs=pl.BlockSpec((TN, B), lambda i: (i, 0)),
    out_shape=jax.ShapeDtypeStruct((VOCAB, B), jnp.float32),
    scratch_shapes=[pltpu.VMEM((3 * EMB, B), jnp.bfloat16)],
    compiler_params=pltpu.CompilerParams(
        dimension_semantics=("arbitrary",),
    ),
)


def kernel(inputs, emb_table, W, b):
    table128 = _tp(emb_table.T)
    idx = inputs.T.reshape(N)
    vecs = _sc_gather(table128, idx)
    vecs3 = vecs.reshape(L, B, PAD)
    logits_t = _mm(vecs3, W.T, b.reshape(1, VOCAB))
    return logits_t.T


# speedup vs baseline: 3.2955x; 1.0089x over previous
import functools

import jax
import jax.numpy as jnp
from jax import lax
from jax.experimental import pallas as pl
from jax.experimental.pallas import tpu as pltpu
from jax.experimental.pallas import tpu_sc as plsc

VOCAB = 100000
EMB = 64
PAD = 128
MAX_NORM = 1.0
B = 1024
L = 20
N = B * L
CHUNK = 128
NCHUNKS = N // CHUNK

_info = plsc.get_sparse_core_info()
NC, NS = _info.num_cores, _info.num_subcores
NW = NC * NS
CPW = NCHUNKS // NW

TN = 2048
GRID = (VOCAB + TN - 1) // TN


def _sc_gather_body(table_hbm, idx_hbm, out_hbm, idx_v, rows_v, sem):
    wid = lax.axis_index("s") * NC + lax.axis_index("c")
    base = wid * CPW
    for j in range(CPW):
        pltpu.sync_copy(
            idx_hbm.at[pl.ds((base + j) * CHUNK, CHUNK)], idx_v.at[j])
    copies = [
        pltpu.async_copy(table_hbm.at[idx_v.at[j]], rows_v.at[j], sem)
        for j in range(CPW)
    ]
    for c in copies:
        c.wait()
    pltpu.sync_copy(rows_v, out_hbm.at[pl.ds(base, CPW)])


_sc_gather = functools.partial(
    pl.kernel,
    mesh=plsc.VectorSubcoreMesh(core_axis_name="c", subcore_axis_name="s"),
    out_type=jax.ShapeDtypeStruct((NCHUNKS, CHUNK, PAD), jnp.float32),
    scratch_types=[
        pltpu.VMEM((CPW, CHUNK), jnp.int32),
        pltpu.VMEM((CPW, CHUNK, PAD), jnp.float32),
        pltpu.SemaphoreType.DMA,
    ],
    compiler_params=pltpu.CompilerParams(use_tc_tiling_on_sc=False),
)(_sc_gather_body)


def _tp_body(embt_ref, out_ref):
    v = embt_ref[...]
    ss = jnp.sum(v * v, axis=0, keepdims=True)
    norm = jnp.sqrt(ss)
    scale = jnp.where(norm > MAX_NORM, MAX_NORM / (norm + 1e-7), 1.0)
    v = v * scale
    vhi = v.astype(jnp.bfloat16)
    vlo = (v - vhi.astype(jnp.float32)).astype(jnp.bfloat16)
    eye = (lax.broadcasted_iota(jnp.int32, (EMB, EMB), 0)
           == lax.broadcasted_iota(jnp.int32, (EMB, EMB), 1))
    ident = eye.astype(jnp.bfloat16)
    vt = lax.dot_general(
        jnp.concatenate([vhi, vlo], axis=0),
        jnp.concatenate([ident, ident], axis=0),
        dimension_numbers=(((0,), (0,)), ((), ())),
        preferred_element_type=jnp.float32,
    )
    z = jnp.zeros((TV, PAD - EMB), jnp.float32)
    out_ref[...] = jnp.concatenate([vt, z], axis=1)


TV = 16384
_tp = pl.pallas_call(
    _tp_body,
    grid=((VOCAB + TV - 1) // TV,),
    in_specs=[pl.BlockSpec((EMB, TV), lambda i: (0, i))],
    out_specs=pl.BlockSpec((TV, PAD), lambda i: (i, 0)),
    out_shape=jax.ShapeDtypeStruct((VOCAB, PAD), jnp.float32),
)


def _mm_body(vecs_ref, wt_ref, b_ref, out_ref, ht_ref):
    @pl.when(pl.program_id(0) == 0)
    def _():
        v = vecs_ref[...]
        h = jnp.sum(v, axis=0) * (1.0 / L)
        ht = h.T[:EMB, :]
        hhi = ht.astype(jnp.bfloat16)
        hlo = (ht - hhi.astype(jnp.float32)).astype(jnp.bfloat16)
        ht_ref[...] = jnp.concatenate([hhi, hhi, hlo], axis=0)

    wt = wt_ref[...]
    whi = wt.astype(jnp.bfloat16)
    wlo = (wt - whi.astype(jnp.float32)).astype(jnp.bfloat16)
    wcat = jnp.concatenate([whi, wlo, whi], axis=0)
    res = lax.dot_general(
        wcat, ht_ref[...],
        dimension_numbers=(((0,), (0,)), ((), ())),
        preferred_element_type=jnp.float32,
    )
    out_ref[...] = res + b_ref[...].T


_mm = pl.pallas_call(
    _mm_body,
    grid=(GRID,),
    in_specs=[
        pl.BlockSpec((L, B, PAD), lambda i: (0, 0, 0)),
        pl.BlockSpec((EMB, TN), lambda i: (0, i)),
        pl.BlockSpec((1, TN), lambda i: (0, i)),
    ],
    out_specs=pl.BlockSpec((TN, B), lambda i: (i, 0)),
    out_shape=jax.ShapeDtypeStruct((VOCAB, B), jnp.float32),
    scratch_shapes=[pltpu.VMEM((3 * EMB, B), jnp.bfloat16)],
    compiler_params=pltpu.CompilerParams(
        dimension_semantics=("arbitrary",),
    ),
)


def kernel(inputs, emb_table, W, b):
    table128 = _tp(emb_table.T)
    idx = inputs.T.reshape(N)
    vecs = _sc_gather(table128, idx)
    vecs3 = vecs.reshape(L, B, PAD)
    logits_t = _mm(vecs3, W.T, b.reshape(1, VOCAB))
    return logits_t.T


# speedup vs baseline: 3.3391x; 1.0132x over previous
import functools

import jax
import jax.numpy as jnp
from jax import lax
from jax.experimental import pallas as pl
from jax.experimental.pallas import tpu as pltpu
from jax.experimental.pallas import tpu_sc as plsc

VOCAB = 100000
EMB = 64
PAD = 128
MAX_NORM = 1.0
B = 1024
L = 20
N = B * L
CHUNK = 128
NCHUNKS = N // CHUNK

_info = plsc.get_sparse_core_info()
NC, NS = _info.num_cores, _info.num_subcores
NW = NC * NS
CPW = NCHUNKS // NW

TN = 2048
GRID = (VOCAB + TN - 1) // TN


def _sc_gather_body(table_hbm, idx_hbm, out_hbm, idx_v, rows_v, sem):
    wid = lax.axis_index("s") * NC + lax.axis_index("c")
    base = wid * CPW
    for j in range(CPW):
        pltpu.sync_copy(
            idx_hbm.at[pl.ds((base + j) * CHUNK, CHUNK)], idx_v.at[j])
    copies = [
        pltpu.async_copy(table_hbm.at[idx_v.at[j]], rows_v.at[j], sem)
        for j in range(CPW)
    ]
    for c in copies:
        c.wait()
    pltpu.sync_copy(rows_v, out_hbm.at[pl.ds(base, CPW)])


_sc_gather = functools.partial(
    pl.kernel,
    mesh=plsc.VectorSubcoreMesh(core_axis_name="c", subcore_axis_name="s"),
    out_type=jax.ShapeDtypeStruct((NCHUNKS, CHUNK, PAD), jnp.float32),
    scratch_types=[
        pltpu.VMEM((CPW, CHUNK), jnp.int32),
        pltpu.VMEM((CPW, CHUNK, PAD), jnp.float32),
        pltpu.SemaphoreType.DMA,
    ],
    compiler_params=pltpu.CompilerParams(use_tc_tiling_on_sc=False),
)(_sc_gather_body)


def _tp_body(embt_ref, out_ref):
    v = embt_ref[...]
    ss = jnp.sum(v * v, axis=0, keepdims=True)
    norm = jnp.sqrt(ss)
    scale = jnp.where(norm > MAX_NORM, MAX_NORM / (norm + 1e-7), 1.0)
    v = v * scale
    vhi = v.astype(jnp.bfloat16)
    vlo = (v - vhi.astype(jnp.float32)).astype(jnp.bfloat16)
    eye = (lax.broadcasted_iota(jnp.int32, (EMB, EMB), 0)
           == lax.broadcasted_iota(jnp.int32, (EMB, EMB), 1))
    ident = eye.astype(jnp.bfloat16)
    vt = lax.dot_general(
        jnp.concatenate([vhi, vlo], axis=0),
        jnp.concatenate([ident, ident], axis=0),
        dimension_numbers=(((0,), (0,)), ((), ())),
        preferred_element_type=jnp.float32,
    )
    z = jnp.zeros((TV, PAD - EMB), jnp.float32)
    out_ref[...] = jnp.concatenate([vt, z], axis=1)


TV = 32768
_tp = pl.pallas_call(
    _tp_body,
    grid=((VOCAB + TV - 1) // TV,),
    in_specs=[pl.BlockSpec((EMB, TV), lambda i: (0, i))],
    out_specs=pl.BlockSpec((TV, PAD), lambda i: (i, 0)),
    out_shape=jax.ShapeDtypeStruct((VOCAB, PAD), jnp.float32),
)


def _mm_body(vecs_ref, wt_ref, b_ref, out_ref, ht_ref):
    @pl.when(pl.program_id(0) == 0)
    def _():
        v = vecs_ref[...]
        h = jnp.sum(v, axis=0) * (1.0 / L)
        ht = h.T[:EMB, :]
        hhi = ht.astype(jnp.bfloat16)
        hlo = (ht - hhi.astype(jnp.float32)).astype(jnp.bfloat16)
        ht_ref[...] = jnp.concatenate([hhi, hhi, hlo], axis=0)

    wt = wt_ref[...]
    whi = wt.astype(jnp.bfloat16)
    wlo = (wt - whi.astype(jnp.float32)).astype(jnp.bfloat16)
    wcat = jnp.concatenate([whi, wlo, whi], axis=0)
    res = lax.dot_general(
        wcat, ht_ref[...],
        dimension_numbers=(((0,), (0,)), ((), ())),
        preferred_element_type=jnp.float32,
    )
    out_ref[...] = res + b_ref[...].T


_mm = pl.pallas_call(
    _mm_body,
    grid=(GRID,),
    in_specs=[
        pl.BlockSpec((L, B, PAD), lambda i: (0, 0, 0)),
        pl.BlockSpec((EMB, TN), lambda i: (0, i)),
        pl.BlockSpec((1, TN), lambda i: (0, i)),
    ],
    out_specs=pl.BlockSpec((TN, B), lambda i: (i, 0)),
    out_shape=jax.ShapeDtypeStruct((VOCAB, B), jnp.float32),
    scratch_shapes=[pltpu.VMEM((3 * EMB, B), jnp.bfloat16)],
    compiler_params=pltpu.CompilerParams(
        dimension_semantics=("arbitrary",),
    ),
)


def kernel(inputs, emb_table, W, b):
    table128 = _tp(emb_table.T)
    idx = inputs.T.reshape(N)
    vecs = _sc_gather(table128, idx)
    vecs3 = vecs.reshape(L, B, PAD)
    logits_t = _mm(vecs3, W.T, b.reshape(1, VOCAB))
    return logits_t.T
